# Initial kernel scaffold; baseline (speedup 1.0000x reference)
import jax
import jax.numpy as jnp
from jax.experimental import pallas as pl


def kernel(x, W):
    raise NotImplementedError("write your pallas kernel here")


# speedup vs baseline: 1.0680x; 1.0680x over previous
import jax
import jax.numpy as jnp
from jax import lax
from jax.experimental import pallas as pl
from jax.experimental.pallas import tpu as pltpu

_TOP_K = 8
_BLOCK = 512


def _router_body(x_ref, wt_ref, idx_ref, w_ref, p_ref):
    logits = jnp.dot(x_ref[...], wt_ref[...], preferred_element_type=jnp.float32)
    m = jnp.max(logits, axis=-1, keepdims=True)
    e = jnp.exp(logits - m)
    probs = e / jnp.sum(e, axis=-1, keepdims=True)
    p_ref[...] = probs
    ne = probs.shape[1]
    col = lax.broadcasted_iota(jnp.int32, probs.shape, 1)
    vals = probs
    idx_cols, w_cols = [], []
    for _ in range(_TOP_K):
        mj = jnp.max(vals, axis=-1, keepdims=True)
        amj = jnp.min(jnp.where(vals == mj, col, ne), axis=-1, keepdims=True)
        w_cols.append(mj)
        idx_cols.append(amj)
        vals = jnp.where(col == amj, -1.0, vals)
    idx_ref[...] = jnp.concatenate(idx_cols, axis=1)
    w_ref[...] = jnp.concatenate(w_cols, axis=1)


@jax.jit
def kernel(x, W):
    n, d = x.shape
    ne = W.shape[0]
    wt = W.T
    out = pl.pallas_call(
        _router_body,
        grid=(n // _BLOCK,),
        in_specs=[
            pl.BlockSpec((_BLOCK, d), lambda i: (i, 0)),
            pl.BlockSpec((d, ne), lambda i: (0, 0)),
        ],
        out_specs=[
            pl.BlockSpec((_BLOCK, _TOP_K), lambda i: (i, 0)),
            pl.BlockSpec((_BLOCK, _TOP_K), lambda i: (i, 0)),
            pl.BlockSpec((_BLOCK, ne), lambda i: (i, 0)),
        ],
        out_shape=[
            jax.ShapeDtypeStruct((n, _TOP_K), jnp.int32),
            jax.ShapeDtypeStruct((n, _TOP_K), jnp.float32),
            jax.ShapeDtypeStruct((n, ne), jnp.float32),
        ],
        compiler_params=pltpu.CompilerParams(
            dimension_semantics=("parallel",)
        ),
    )(x, wt)
    indices, weights, probs = out
    return (indices, weights, probs)


# speedup vs baseline: 1.2100x; 1.1329x over previous
import jax
import jax.numpy as jnp
from jax import lax
from jax.experimental import pallas as pl
from jax.experimental.pallas import tpu as pltpu

_TOP_K = 8
_BLOCK = 1024


def _router_body(x_ref, wt_ref, idx_ref, w_ref, p_ref):
    logits = jnp.dot(x_ref[...], wt_ref[...], preferred_element_type=jnp.float32)
    m = jnp.max(logits, axis=-1, keepdims=True)
    e = jnp.exp(logits - m)
    probs = e / jnp.sum(e, axis=-1, keepdims=True)
    p_ref[...] = probs
    ne = probs.shape[1]
    col = lax.broadcasted_iota(jnp.int32, probs.shape, 1)
    vals = probs
    idx_cols, w_cols = [], []
    for _ in range(_TOP_K):
        mj = jnp.max(vals, axis=-1, keepdims=True)
        amj = jnp.min(jnp.where(vals == mj, col, ne), axis=-1, keepdims=True)
        w_cols.append(mj)
        idx_cols.append(amj)
        vals = jnp.where(col == amj, -1.0, vals)
    idx_ref[...] = jnp.concatenate(idx_cols, axis=1)
    w_ref[...] = jnp.concatenate(w_cols, axis=1)


@jax.jit
def kernel(x, W):
    n, d = x.shape
    ne = W.shape[0]
    wt = W.T
    out = pl.pallas_call(
        _router_body,
        grid=(n // _BLOCK,),
        in_specs=[
            pl.BlockSpec((_BLOCK, d), lambda i: (i, 0)),
            pl.BlockSpec((d, ne), lambda i: (0, 0)),
        ],
        out_specs=[
            pl.BlockSpec((_BLOCK, _TOP_K), lambda i: (i, 0)),
            pl.BlockSpec((_BLOCK, _TOP_K), lambda i: (i, 0)),
            pl.BlockSpec((_BLOCK, ne), lambda i: (i, 0)),
        ],
        out_shape=[
            jax.ShapeDtypeStruct((n, _TOP_K), jnp.int32),
            jax.ShapeDtypeStruct((n, _TOP_K), jnp.float32),
            jax.ShapeDtypeStruct((n, ne), jnp.float32),
        ],
        compiler_params=pltpu.CompilerParams(
            dimension_semantics=("parallel",)
        ),
    )(x, wt)
    indices, weights, probs = out
    return (indices, weights, probs)


# speedup vs baseline: 1.2141x; 1.0034x over previous
import jax
import jax.numpy as jnp
from jax import lax
from jax.experimental import pallas as pl
from jax.experimental.pallas import tpu as pltpu

_TOP_K = 8
_BLOCK = 1024


def _router_body(xa_ref, xb_ref, wta_ref, wtb_ref, idx_ref, w_ref, p_ref):
    logits = jnp.dot(xa_ref[...], wta_ref[...], preferred_element_type=jnp.float32)
    logits = logits + jnp.dot(xb_ref[...], wtb_ref[...], preferred_element_type=jnp.float32)
    m = jnp.max(logits, axis=-1, keepdims=True)
    e = jnp.exp(logits - m)
    probs = e / jnp.sum(e, axis=-1, keepdims=True)
    p_ref[...] = probs
    ne = probs.shape[1]
    col = lax.broadcasted_iota(jnp.int32, probs.shape, 1)
    vals = probs
    idx_cols, w_cols = [], []
    for _ in range(_TOP_K):
        mj = jnp.max(vals, axis=-1, keepdims=True)
        amj = jnp.min(jnp.where(vals == mj, col, ne), axis=-1, keepdims=True)
        w_cols.append(mj)
        idx_cols.append(amj)
        vals = jnp.where(col == amj, -1.0, vals)
    idx_ref[...] = jnp.concatenate(idx_cols, axis=1)
    w_ref[...] = jnp.concatenate(w_cols, axis=1)


@jax.jit
def kernel(x, W):
    n, d = x.shape
    ne = W.shape[0]
    dh = d // 2
    wt = W.T
    out = pl.pallas_call(
        _router_body,
        grid=(n // _BLOCK,),
        in_specs=[
            pl.BlockSpec((_BLOCK, dh), lambda i: (i, 0)),
            pl.BlockSpec((_BLOCK, dh), lambda i: (i, 1)),
            pl.BlockSpec((dh, ne), lambda i: (0, 0)),
            pl.BlockSpec((dh, ne), lambda i: (1, 0)),
        ],
        out_specs=[
            pl.BlockSpec((_BLOCK, _TOP_K), lambda i: (i, 0)),
            pl.BlockSpec((_BLOCK, _TOP_K), lambda i: (i, 0)),
            pl.BlockSpec((_BLOCK, ne), lambda i: (i, 0)),
        ],
        out_shape=[
            jax.ShapeDtypeStruct((n, _TOP_K), jnp.int32),
            jax.ShapeDtypeStruct((n, _TOP_K), jnp.float32),
            jax.ShapeDtypeStruct((n, ne), jnp.float32),
        ],
        compiler_params=pltpu.CompilerParams(
            dimension_semantics=("parallel",)
        ),
    )(x, x, wt, wt)
    indices, weights, probs = out
    return (indices, weights, probs)
